# baseline (speedup 1.0000x reference)
import functools

import jax
import jax.numpy as jnp
from jax import lax
from jax.experimental import pallas as pl
from jax.experimental.pallas import tpu as pltpu
from jax.experimental.pallas import tpu_sc as plsc

_NC = 2
_NS = 16
_NW = _NC * _NS


def kernel(query, value, weight):
    Lq = query.shape[1]
    Lv = value.shape[1]
    V, D = weight.shape
    S = 4096
    Vp = 1040
    groups = 8
    rows_per_gtile = Lq // _NW
    base = 1528
    JT = Lv // 128
    CH = D // 8

    mesh = plsc.VectorSubcoreMesh(core_axis_name="c", subcore_axis_name="s")

    @functools.partial(
        pl.kernel,
        mesh=mesh,
        out_type=jax.ShapeDtypeStruct((Lq, CH, JT, 8, 128), jnp.float32),
        scratch_types=[
            pltpu.VMEM((CH, 8, S), jnp.float32),
            pltpu.SemaphoreType.DMA,
        ],
        compiler_params=pltpu.CompilerParams(use_tc_tiling_on_sc=False),
    )
    def k(wt_hbm, out_hbm, et_ref, sem):
        wid = lax.axis_index("s") * _NC + lax.axis_index("c")
        g = wid % groups
        q = wid // groups
        r = (groups - 1) - g

        for c in range(D):
            pltpu.async_copy(
                wt_hbm.at[pl.ds((r * D + c) * Vp, Vp)],
                et_ref.at[c // 8, c % 8, pl.ds(base, Vp)],
                sem,
            )
        for c in range(D):
            pltpu.make_async_copy(
                wt_hbm.at[pl.ds((r * D + c) * Vp, Vp)],
                et_ref.at[c // 8, c % 8, pl.ds(base, Vp)],
                sem,
            ).wait()

        hsplat = [
            jnp.full(
                (16,), et_ref[c // 8, c % 8, pl.ds(base, 16)][0], jnp.float32
            )
            for c in range(D)
        ]

        def fill_head(u, _):
            for c in range(D):
                et_ref[c // 8, c % 8, pl.ds(u * 16, 16)] = hsplat[c]
            return 0

        lax.fori_loop(0, base // 16, fill_head, 0)
        for c in range(D):
            et_ref[c // 8, c % 8, pl.ds(base - 16, 16)] = hsplat[c]

        tail0 = base + Vp
        tsplat = [
            jnp.full(
                (16,),
                et_ref[c // 8, c % 8, pl.ds(tail0 - 16, 16)][15],
                jnp.float32,
            )
            for c in range(D)
        ]

        def fill_tail(u, _):
            for c in range(D):
                et_ref[c // 8, c % 8, pl.ds(tail0 + u * 16, 16)] = tsplat[c]
            return 0

        lax.fori_loop(0, (S - tail0) // 16, fill_tail, 0)
        for c in range(D):
            et_ref[c // 8, c % 8, pl.ds(S - 16, 16)] = tsplat[c]

        def fire(m):
            kk = rows_per_gtile * q + m
            i = g + 8 * kk
            t0 = (Lq - 8) - 8 * kk
            for jt in range(JT):
                pltpu.async_copy(
                    et_ref.at[:, :, pl.ds(t0 + 128 * jt, 128)],
                    out_hbm.at[i, :, jt],
                    sem,
                )

        def drain(m):
            kk = rows_per_gtile * q + m
            i = g + 8 * kk
            t0 = (Lq - 8) - 8 * kk
            for jt in range(JT):
                pltpu.make_async_copy(
                    et_ref.at[:, :, pl.ds(t0 + 128 * jt, 128)],
                    out_hbm.at[i, :, jt],
                    sem,
                ).wait()

        fire(0)

        def body(m, _):
            fire(m + 1)
            drain(m)
            return 0

        lax.fori_loop(0, rows_per_gtile - 1, body, 0)
        drain(rows_per_gtile - 1)

    cols = weight.T
    pieces = []
    for r in range(groups):
        for c in range(D):
            f = (groups - 1) - r
            if f:
                pieces.append(jnp.broadcast_to(cols[c, 0], (f,)))
            pieces.append(cols[c])
            pieces.append(jnp.broadcast_to(cols[c, V - 1], (Vp - V - f,)))
    wt_all = jnp.concatenate(pieces)

    out = k(wt_all)
    return out.transpose(0, 2, 4, 1, 3).reshape(Lq, Lv, D)


# speedup vs baseline: 1.9616x; 1.9616x over previous
import functools

import jax
import jax.numpy as jnp
from jax import lax
from jax.experimental import pallas as pl
from jax.experimental.pallas import tpu as pltpu
from jax.experimental.pallas import tpu_sc as plsc

_NC = 2
_NS = 16
_NW = _NC * _NS


def kernel(query, value, weight):
    Lq = query.shape[1]
    Lv = value.shape[1]
    V, D = weight.shape
    S = 4096
    groups = 8
    rows_per_gtile = Lq // _NW
    head = Lq - 1 - (V - 1) // 2
    JT = Lv // 128
    CH = D // 8

    mesh = plsc.VectorSubcoreMesh(core_axis_name="c", subcore_axis_name="s")

    @functools.partial(
        pl.kernel,
        mesh=mesh,
        out_type=jax.ShapeDtypeStruct((Lq, CH, JT, 8, 128), jnp.float32),
        scratch_types=[
            pltpu.VMEM((CH, 8, S), jnp.float32),
            pltpu.SemaphoreType.DMA,
        ],
        compiler_params=pltpu.CompilerParams(use_tc_tiling_on_sc=False),
    )
    def k(wt_hbm, out_hbm, et_ref, sem):
        wid = lax.axis_index("s") * _NC + lax.axis_index("c")
        g = wid % groups
        q = wid // groups
        r = (groups - 1) - g

        for c in range(D):
            pltpu.async_copy(
                wt_hbm.at[c, r], et_ref.at[c // 8, c % 8], sem
            )
        for c in range(D):
            pltpu.make_async_copy(
                wt_hbm.at[c, r], et_ref.at[c // 8, c % 8], sem
            ).wait()

        def fire(m):
            kk = rows_per_gtile * q + m
            i = g + 8 * kk
            t0 = (Lq - 8) - 8 * kk
            for jt in range(JT):
                pltpu.async_copy(
                    et_ref.at[:, :, pl.ds(t0 + 128 * jt, 128)],
                    out_hbm.at[i, :, jt],
                    sem,
                )

        def drain(m):
            kk = rows_per_gtile * q + m
            i = g + 8 * kk
            t0 = (Lq - 8) - 8 * kk
            for jt in range(JT):
                pltpu.make_async_copy(
                    et_ref.at[:, :, pl.ds(t0 + 128 * jt, 128)],
                    out_hbm.at[i, :, jt],
                    sem,
                ).wait()

        fire(0)

        def body(m, _):
            fire(m + 1)
            drain(m)
            return 0

        lax.fori_loop(0, rows_per_gtile - 1, body, 0)
        drain(rows_per_gtile - 1)

    cols = weight.T
    F = jnp.concatenate(
        [
            jnp.broadcast_to(cols[:, :1], (D, head)),
            cols,
            jnp.broadcast_to(cols[:, -1:], (D, S + groups - 1 - head - V)),
        ],
        axis=1,
    )
    eye = jnp.eye(groups, dtype=jnp.float32)[:, None, :]
    wt = lax.conv_general_dilated(
        F[:, None, :],
        eye,
        window_strides=(1,),
        padding="VALID",
        dimension_numbers=("NCH", "OIH", "NCH"),
        precision=lax.Precision.HIGHEST,
    )

    out = k(wt)
    return out.transpose(0, 2, 4, 1, 3).reshape(Lq, Lv, D)


# speedup vs baseline: 2.5200x; 1.2847x over previous
import functools

import jax
import jax.numpy as jnp
from jax import lax
from jax.experimental import pallas as pl
from jax.experimental.pallas import tpu as pltpu
from jax.experimental.pallas import tpu_sc as plsc

_NC = 2
_NS = 16
_NW = _NC * _NS


def kernel(query, value, weight):
    Lq = query.shape[1]
    Lv = value.shape[1]
    V, D = weight.shape
    S = 4096
    Vp = 1040
    groups = 8
    rows_per_gtile = Lq // _NW
    base = 1528
    JT = Lv // 128
    CH = D // 8

    mesh = plsc.VectorSubcoreMesh(core_axis_name="c", subcore_axis_name="s")

    @functools.partial(
        pl.kernel,
        mesh=mesh,
        out_type=jax.ShapeDtypeStruct((Lq, CH, JT, 8, 128), jnp.float32),
        scratch_types=[
            pltpu.VMEM((CH, 8, S), jnp.float32),
            pltpu.SemaphoreType.DMA,
        ],
        compiler_params=pltpu.CompilerParams(use_tc_tiling_on_sc=False),
    )
    def k(wt_hbm, out_hbm, et_ref, sem):
        wid = lax.axis_index("s") * _NC + lax.axis_index("c")
        g = wid % groups
        q = wid // groups
        r = (groups - 1) - g

        for c in range(D):
            pltpu.async_copy(
                wt_hbm.at[pl.ds((r * D + c) * Vp, Vp)],
                et_ref.at[c // 8, c % 8, pl.ds(base, Vp)],
                sem,
            )
        for c in range(D):
            pltpu.make_async_copy(
                wt_hbm.at[pl.ds((r * D + c) * Vp, Vp)],
                et_ref.at[c // 8, c % 8, pl.ds(base, Vp)],
                sem,
            ).wait()

        hsplat = [
            jnp.full(
                (16,), et_ref[c // 8, c % 8, pl.ds(base, 16)][0], jnp.float32
            )
            for c in range(D)
        ]

        def fill_head(u, _):
            for c in range(D):
                et_ref[c // 8, c % 8, pl.ds(u * 16, 16)] = hsplat[c]
            return 0

        ulo = jnp.minimum(96 - 32 * q, base // 16)
        lax.fori_loop(ulo, base // 16, fill_head, 0)
        for c in range(D):
            et_ref[c // 8, c % 8, pl.ds(base - 16, 16)] = hsplat[c]

        tail0 = base + Vp
        tsplat = [
            jnp.full(
                (16,),
                et_ref[c // 8, c % 8, pl.ds(tail0 - 16, 16)][15],
                jnp.float32,
            )
            for c in range(D)
        ]

        def fill_tail(u, _):
            for c in range(D):
                et_ref[c // 8, c % 8, pl.ds(tail0 + u * 16, 16)] = tsplat[c]
            return 0

        uhi = jnp.maximum(95 - 32 * q, 0)
        lax.fori_loop(0, uhi, fill_tail, 0)
        for c in range(D):
            et_ref[c // 8, c % 8, pl.ds(S - 16, 16)] = tsplat[c]

        def fire(m):
            kk = rows_per_gtile * q + m
            i = g + 8 * kk
            t0 = (Lq - 8) - 8 * kk
            for jt in range(JT):
                pltpu.async_copy(
                    et_ref.at[:, :, pl.ds(t0 + 128 * jt, 128)],
                    out_hbm.at[i, :, jt],
                    sem,
                )

        def drain(m):
            kk = rows_per_gtile * q + m
            i = g + 8 * kk
            t0 = (Lq - 8) - 8 * kk
            for jt in range(JT):
                pltpu.make_async_copy(
                    et_ref.at[:, :, pl.ds(t0 + 128 * jt, 128)],
                    out_hbm.at[i, :, jt],
                    sem,
                ).wait()

        fire(0)

        def body(m, _):
            fire(m + 1)
            drain(m)
            return 0

        lax.fori_loop(0, rows_per_gtile - 1, body, 0)
        drain(rows_per_gtile - 1)

    cols = weight.T
    w0 = cols[:, :1]
    wl = cols[:, -1:]
    bands = [
        jnp.concatenate(
            [
                jnp.repeat(w0, (groups - 1) - r, axis=1),
                cols,
                jnp.repeat(wl, Vp - V - ((groups - 1) - r), axis=1),
            ],
            axis=1,
        )
        for r in range(groups)
    ]
    wt_all = jnp.stack(bands).reshape(groups * D * Vp)

    out = k(wt_all)
    return out.transpose(0, 2, 4, 1, 3).reshape(Lq, Lv, D)
